# baseline (speedup 1.0000x reference)
import functools

import jax
import jax.numpy as jnp
from jax import lax
from jax.experimental import pallas as pl
from jax.experimental.pallas import tpu as pltpu
from jax.experimental.pallas import tpu_sc as plsc

_NC = 2
_NS = 16
_NW = _NC * _NS
_LANES = 16

_CHUNK_ROWS = 32


def _make_sc_add(B, S, D):
    R = S // _NW
    C = _CHUNK_ROWS
    CD = C * D
    NCH = R // C
    mesh = plsc.VectorSubcoreMesh(core_axis_name="c", subcore_axis_name="s")

    @functools.partial(
        pl.kernel,
        mesh=mesh,
        out_type=jax.ShapeDtypeStruct((B * S * D,), jnp.float32),
        scratch_types=[
            pltpu.VMEM((CD,), jnp.float32),
            pltpu.VMEM((CD,), jnp.float32),
        ],
    )
    def sc_add(tok_hbm, tab_hbm, out_hbm, tok_v, tab_v):
        wid = lax.axis_index("s") * _NC + lax.axis_index("c")
        base = wid * R * D
        for c in range(NCH):
            tab_off = base + c * CD
            pltpu.sync_copy(tab_hbm.at[pl.ds(tab_off, CD)], tab_v)
            for b in range(B):
                tok_off = b * S * D + tab_off
                pltpu.sync_copy(tok_hbm.at[pl.ds(tok_off, CD)], tok_v)

                def add_body(i, _):
                    o = i * _LANES
                    tok_v[pl.ds(o, _LANES)] = (
                        tok_v[pl.ds(o, _LANES)] + tab_v[pl.ds(o, _LANES)]
                    )
                    return 0

                lax.fori_loop(0, CD // _LANES, add_body, 0)
                pltpu.sync_copy(tok_v, out_hbm.at[pl.ds(tok_off, CD)])

    return sc_add


def kernel(encoded_tokens, pos_table):
    B, S, D = encoded_tokens.shape
    sc_add = _make_sc_add(B, S, D)
    out_flat = sc_add(
        encoded_tokens.reshape(B * S * D), pos_table.reshape(S * D)
    )
    return out_flat.reshape(B, S, D)


# speedup vs baseline: 1.0308x; 1.0308x over previous
import functools

import jax
import jax.numpy as jnp
from jax import lax
from jax.experimental import pallas as pl
from jax.experimental.pallas import tpu as pltpu
from jax.experimental.pallas import tpu_sc as plsc

_NC = 2
_NS = 16
_NW = _NC * _NS
_LANES = 16

_CHUNK_ROWS = 16
_NBUF = 4
_UNROLL = 16


def _make_sc_add(B, S, D):
    R = S // _NW
    C = _CHUNK_ROWS
    CD = C * D
    NCH = R // C
    NSTEP = NCH * B
    mesh = plsc.VectorSubcoreMesh(core_axis_name="c", subcore_axis_name="s")

    @functools.partial(
        pl.kernel,
        mesh=mesh,
        out_type=jax.ShapeDtypeStruct((B * S * D,), jnp.float32),
        scratch_types=[
            pltpu.VMEM((_NBUF, CD), jnp.float32),
            pltpu.VMEM((2, CD), jnp.float32),
            pltpu.SemaphoreType.DMA,
            pltpu.SemaphoreType.DMA,
            pltpu.SemaphoreType.DMA,
        ],
    )
    def sc_add(tok_hbm, tab_hbm, out_hbm, tok_v, tab_v, in_sem, out_sem, tab_sem):
        wid = lax.axis_index("s") * _NC + lax.axis_index("c")
        base = wid * R * D

        def tok_off(step):
            c, b = divmod(step, B)
            return b * S * D + base + c * CD

        tab_waits = [
            pltpu.async_copy(tab_hbm.at[pl.ds(base, CD)], tab_v.at[0], tab_sem)
        ]
        in_waits = [
            pltpu.async_copy(
                tok_hbm.at[pl.ds(tok_off(k), CD)], tok_v.at[k % _NBUF], in_sem
            )
            for k in range(min(_NBUF, NSTEP))
        ]
        out_waits = []

        for step in range(NSTEP):
            c, b = divmod(step, B)
            buf = step % _NBUF
            tbuf = c % 2
            if b == 0:
                tab_waits.pop(0).wait()
            in_waits.pop(0).wait()

            def add_body(i, _, buf=buf, tbuf=tbuf):
                o = i * (_LANES * _UNROLL)
                for u in range(_UNROLL):
                    sl = pl.ds(o + u * _LANES, _LANES)
                    tok_v[buf, sl] = tok_v[buf, sl] + tab_v[tbuf, sl]
                return 0

            lax.fori_loop(0, CD // (_LANES * _UNROLL), add_body, 0)

            out_waits.append(
                pltpu.async_copy(
                    tok_v.at[buf], out_hbm.at[pl.ds(tok_off(step), CD)], out_sem
                )
            )
            if b == B - 1 and c + 1 < NCH:
                tab_waits.append(
                    pltpu.async_copy(
                        tab_hbm.at[pl.ds(base + (c + 1) * CD, CD)],
                        tab_v.at[(c + 1) % 2],
                        tab_sem,
                    )
                )
            nxt = step + _NBUF
            if nxt < NSTEP:
                if len(out_waits) > _NBUF - 1:
                    out_waits.pop(0).wait()
                in_waits.append(
                    pltpu.async_copy(
                        tok_hbm.at[pl.ds(tok_off(nxt), CD)], tok_v.at[buf], in_sem
                    )
                )
        for w in out_waits:
            w.wait()

    return sc_add


def kernel(encoded_tokens, pos_table):
    B, S, D = encoded_tokens.shape
    sc_add = _make_sc_add(B, S, D)
    out_flat = sc_add(
        encoded_tokens.reshape(B * S * D), pos_table.reshape(S * D)
    )
    return out_flat.reshape(B, S, D)


# speedup vs baseline: 1.5014x; 1.4566x over previous
import functools

import jax
import jax.numpy as jnp
from jax import lax
from jax.experimental import pallas as pl
from jax.experimental.pallas import tpu as pltpu
from jax.experimental.pallas import tpu_sc as plsc

_NC = 2
_NS = 16
_NW = _NC * _NS
_LANES = 16

_CHUNK_ROWS = 16
_NBUF = 4
_UNROLL = 8


def _make_sc_add(B, S, D):
    R = S // _NW
    C = _CHUNK_ROWS
    CD = C * D
    NCH = R // C
    NSTEP = NCH * B
    mesh = plsc.VectorSubcoreMesh(core_axis_name="c", subcore_axis_name="s")

    @functools.partial(
        pl.kernel,
        mesh=mesh,
        out_type=jax.ShapeDtypeStruct((B * S * D,), jnp.float32),
        scratch_types=[
            pltpu.VMEM((_NBUF, CD), jnp.float32),
            pltpu.VMEM((2, CD), jnp.float32),
            pltpu.SemaphoreType.DMA,
            pltpu.SemaphoreType.DMA,
            pltpu.SemaphoreType.DMA,
        ],
    )
    def sc_add(tok_hbm, tab_hbm, out_hbm, tok_v, tab_v, in_sem, out_sem, tab_sem):
        wid = lax.axis_index("s") * _NC + lax.axis_index("c")
        base = wid * R * D

        def tok_off(step):
            c, b = divmod(step, B)
            return b * S * D + base + c * CD

        tab_waits = [
            pltpu.async_copy(tab_hbm.at[pl.ds(base, CD)], tab_v.at[0], tab_sem)
        ]
        in_waits = [
            pltpu.async_copy(
                tok_hbm.at[pl.ds(tok_off(k), CD)], tok_v.at[k % _NBUF], in_sem
            )
            for k in range(min(_NBUF, NSTEP))
        ]
        out_waits = []

        for step in range(NSTEP):
            c, b = divmod(step, B)
            buf = step % _NBUF
            tbuf = c % 2
            if b == 0:
                tab_waits.pop(0).wait()
            in_waits.pop(0).wait()

            @plsc.parallel_loop(0, CD, step=_LANES, unroll=_UNROLL)
            def add_body(o, buf=buf, tbuf=tbuf):
                sl = pl.ds(o, _LANES)
                tok_v[buf, sl] = tok_v[buf, sl] + tab_v[tbuf, sl]

            out_waits.append(
                pltpu.async_copy(
                    tok_v.at[buf], out_hbm.at[pl.ds(tok_off(step), CD)], out_sem
                )
            )
            if b == B - 1 and c + 1 < NCH:
                tab_waits.append(
                    pltpu.async_copy(
                        tab_hbm.at[pl.ds(base + (c + 1) * CD, CD)],
                        tab_v.at[(c + 1) % 2],
                        tab_sem,
                    )
                )
            nxt = step + _NBUF
            if nxt < NSTEP:
                if len(out_waits) > _NBUF - 1:
                    out_waits.pop(0).wait()
                in_waits.append(
                    pltpu.async_copy(
                        tok_hbm.at[pl.ds(tok_off(nxt), CD)], tok_v.at[buf], in_sem
                    )
                )
        for w in out_waits:
            w.wait()

    return sc_add


def kernel(encoded_tokens, pos_table):
    B, S, D = encoded_tokens.shape
    sc_add = _make_sc_add(B, S, D)
    out_flat = sc_add(
        encoded_tokens.reshape(B * S * D), pos_table.reshape(S * D)
    )
    return out_flat.reshape(B, S, D)


# speedup vs baseline: 5.0428x; 3.3588x over previous
import functools

import jax
import jax.numpy as jnp
from jax import lax
from jax.experimental import pallas as pl
from jax.experimental.pallas import tpu as pltpu
from jax.experimental.pallas import tpu_sc as plsc

_NC = 2
_NS = 16
_NW = _NC * _NS
_LANES = 16

_CHUNK_ROWS = 32
_NBUF = 3
_UNROLL = 8


def _make_sc_add(B, S, D):
    R = S // _NW
    C = _CHUNK_ROWS
    NCH = R // C
    NSTEP = NCH * B
    SL = D // _LANES
    mesh = plsc.VectorSubcoreMesh(core_axis_name="c", subcore_axis_name="s")

    @functools.partial(
        pl.kernel,
        mesh=mesh,
        out_type=jax.ShapeDtypeStruct((B, S, D), jnp.float32),
        compiler_params=pltpu.CompilerParams(use_tc_tiling_on_sc=True),
        scratch_types=[
            pltpu.VMEM((_NBUF, C, D), jnp.float32),
            pltpu.VMEM((2, C, D), jnp.float32),
            pltpu.SemaphoreType.DMA,
            pltpu.SemaphoreType.DMA,
            pltpu.SemaphoreType.DMA,
        ],
    )
    def sc_add(tok_hbm, tab_hbm, out_hbm, tok_v, tab_v, in_sem, out_sem, tab_sem):
        wid = lax.axis_index("s") * _NC + lax.axis_index("c")
        row0 = wid * R

        def step_rows(step):
            c, b = divmod(step, B)
            return b, row0 + c * C

        tab_waits = [
            pltpu.async_copy(tab_hbm.at[pl.ds(row0, C)], tab_v.at[0], tab_sem)
        ]
        in_waits = []
        for k in range(min(_NBUF, NSTEP)):
            b, r = step_rows(k)
            in_waits.append(
                pltpu.async_copy(
                    tok_hbm.at[b, pl.ds(r, C)], tok_v.at[k % _NBUF], in_sem
                )
            )
        out_waits = []

        for step in range(NSTEP):
            c, b = divmod(step, B)
            buf = step % _NBUF
            tbuf = c % 2
            if b == 0:
                tab_waits.pop(0).wait()
            in_waits.pop(0).wait()

            @plsc.parallel_loop(0, C * SL, unroll=_UNROLL)
            def add_body(i, buf=buf, tbuf=tbuf):
                r = i // SL
                col = (i - r * SL) * _LANES
                sl = pl.ds(col, _LANES)
                tok_v[buf, r, sl] = tok_v[buf, r, sl] + tab_v[tbuf, r, sl]

            _, rows = step_rows(step)
            out_waits.append(
                pltpu.async_copy(
                    tok_v.at[buf], out_hbm.at[b, pl.ds(rows, C)], out_sem
                )
            )
            if b == B - 1 and c + 1 < NCH:
                tab_waits.append(
                    pltpu.async_copy(
                        tab_hbm.at[pl.ds(row0 + (c + 1) * C, C)],
                        tab_v.at[(c + 1) % 2],
                        tab_sem,
                    )
                )
            nxt = step + _NBUF
            if nxt < NSTEP:
                if len(out_waits) > _NBUF - 1:
                    out_waits.pop(0).wait()
                nb, nr = step_rows(nxt)
                in_waits.append(
                    pltpu.async_copy(
                        tok_hbm.at[nb, pl.ds(nr, C)], tok_v.at[buf], in_sem
                    )
                )
        for w in out_waits:
            w.wait()

    return sc_add


def kernel(encoded_tokens, pos_table):
    B, S, D = encoded_tokens.shape
    sc_add = _make_sc_add(B, S, D)
    return sc_add(encoded_tokens, pos_table)


# speedup vs baseline: 5.2951x; 1.0500x over previous
import functools

import jax
import jax.numpy as jnp
from jax import lax
from jax.experimental import pallas as pl
from jax.experimental.pallas import tpu as pltpu
from jax.experimental.pallas import tpu_sc as plsc

_NC = 2
_NS = 16
_NW = _NC * _NS
_LANES = 16

_CHUNK_ROWS = 8
_NGRP = 4
_PREF = 3
_UNROLL = 4


def _make_sc_add(B, S, D):
    R = S // _NW
    C = _CHUNK_ROWS
    NCH = R // C
    SL = D // _LANES
    mesh = plsc.VectorSubcoreMesh(core_axis_name="c", subcore_axis_name="s")

    @functools.partial(
        pl.kernel,
        mesh=mesh,
        out_type=jax.ShapeDtypeStruct((B, S, D), jnp.float32),
        compiler_params=pltpu.CompilerParams(use_tc_tiling_on_sc=True),
        scratch_types=[
            pltpu.VMEM((_NGRP, B, C, D), jnp.float32),
            pltpu.VMEM((_NGRP, C, D), jnp.float32),
            pltpu.SemaphoreType.DMA,
            pltpu.SemaphoreType.DMA,
            pltpu.SemaphoreType.DMA,
        ],
    )
    def sc_add(tok_hbm, tab_hbm, out_hbm, tok_v, tab_v, in_sem, out_sem, tab_sem):
        wid = lax.axis_index("s") * _NC + lax.axis_index("c")
        row0 = wid * R

        def start_in(c):
            g = c % _NGRP
            rows = row0 + c * C
            waits = [
                pltpu.async_copy(
                    tab_hbm.at[pl.ds(rows, C)], tab_v.at[g], tab_sem
                )
            ]
            for b in range(B):
                waits.append(
                    pltpu.async_copy(
                        tok_hbm.at[b, pl.ds(rows, C)], tok_v.at[g, b], in_sem
                    )
                )
            return waits

        in_waits = []
        out_waits = []
        for c in range(min(_PREF, NCH)):
            in_waits.append(start_in(c))

        for c in range(NCH):
            g = c % _NGRP
            rows = row0 + c * C
            for w in in_waits.pop(0):
                w.wait()

            @plsc.parallel_loop(0, C * SL, unroll=_UNROLL)
            def add_body(i, g=g):
                r = i // SL
                col = (i - r * SL) * _LANES
                sl = pl.ds(col, _LANES)
                t = tab_v[g, r, sl]
                for b in range(B):
                    tok_v[g, b, r, sl] = tok_v[g, b, r, sl] + t

            out_waits.append(
                [
                    pltpu.async_copy(
                        tok_v.at[g, b], out_hbm.at[b, pl.ds(rows, C)], out_sem
                    )
                    for b in range(B)
                ]
            )
            nxt = c + _PREF
            if nxt < NCH:
                if len(out_waits) > _NGRP - _PREF:
                    for w in out_waits.pop(0):
                        w.wait()
                in_waits.append(start_in(nxt))
        for ws in out_waits:
            for w in ws:
                w.wait()

    return sc_add


def kernel(encoded_tokens, pos_table):
    B, S, D = encoded_tokens.shape
    sc_add = _make_sc_add(B, S, D)
    return sc_add(encoded_tokens, pos_table)
